# baseline (speedup 1.0000x reference)
import jax
import jax.numpy as jnp
from jax import lax
from jax.experimental import pallas as pl
from jax.experimental.pallas import tpu as pltpu
from jax.experimental.pallas import tpu_sc as plsc

N = 10000
EDGES = 320000
F_IN = 128
F_H = 64

N_PAD = 10240
N_PER_TILE = N_PAD // 16
CHUNK = 128
N_TILES = 32
NCHUNKS = EDGES // CHUNK
CPT = 80
LAST_RING = 16
LAST_TAIL = NCHUNKS - 31 * CPT - LAST_RING
ROW_BLK = 1000

NBUF = 8
PF = 4


def _mesh():
    return plsc.VectorSubcoreMesh(core_axis_name="c", subcore_axis_name="s")


def _stage_indices(src_h, dst_h, srcv, dstv, tid):
    @pl.when(tid < 31)
    def _full():
        pltpu.sync_copy(src_h.at[pl.ds(tid * CPT, CPT)], srcv.at[pl.ds(0, CPT)])
        pltpu.sync_copy(dst_h.at[pl.ds(tid * CPT, CPT)], dstv.at[pl.ds(0, CPT)])

    @pl.when(tid == 31)
    def _last():
        nlast = LAST_RING + LAST_TAIL
        pltpu.sync_copy(src_h.at[pl.ds(31 * CPT, nlast)], srcv.at[pl.ds(0, nlast)])
        pltpu.sync_copy(dst_h.at[pl.ds(31 * CPT, nlast)], dstv.at[pl.ds(0, nlast)])


def _sc_degrees(src2d, dst2d):
    def body(src_h, dst_h, dout_h, din_h, srcv, dstv, ones_v, zv, acc_o, acc_i,
             sem_a, sem_b):
        c = lax.axis_index("c")
        s = lax.axis_index("s")
        tid = s * 2 + c
        nt = jnp.where(tid < 31, CPT, LAST_RING + LAST_TAIL)

        def set_ones(i, _):
            ones_v[pl.ds(i * 16, 16)] = jnp.ones((16,), jnp.float32)
            return 0

        lax.fori_loop(0, CHUNK // 16, set_ones, 0)

        def set_zero(i, _):
            zv[pl.ds(i * 16, 16)] = jnp.zeros((16,), jnp.float32)
            return 0

        lax.fori_loop(0, N_PER_TILE // 16, set_zero, 0)

        sl = pl.ds(s * N_PER_TILE, N_PER_TILE)
        pltpu.sync_copy(zv, acc_o.at[sl])
        pltpu.sync_copy(zv, acc_i.at[sl])
        plsc.subcore_barrier()

        _stage_indices(src_h, dst_h, srcv, dstv, tid)

        def step(j, _):
            pltpu.async_copy(ones_v, acc_o.at[srcv.at[j]], sem_a, add=True)
            pltpu.async_copy(ones_v, acc_i.at[dstv.at[j]], sem_b, add=True)

            @pl.when(j >= 16)
            def _drain_old():
                pltpu.make_async_copy(ones_v, acc_o.at[srcv.at[j - 16]], sem_a).wait()
                pltpu.make_async_copy(ones_v, acc_i.at[dstv.at[j - 16]], sem_b).wait()

            return 0

        lax.fori_loop(0, nt, step, 0)

        def drain(i, _):
            pltpu.make_async_copy(ones_v, acc_o.at[srcv.at[nt - 16 + i]], sem_a).wait()
            pltpu.make_async_copy(ones_v, acc_i.at[dstv.at[nt - 16 + i]], sem_b).wait()
            return 0

        lax.fori_loop(0, 16, drain, 0)
        plsc.subcore_barrier()

        pltpu.sync_copy(acc_o.at[sl], dout_h.at[c, sl])
        pltpu.sync_copy(acc_i.at[sl], din_h.at[c, sl])

    return pl.kernel(
        body,
        out_type=[
            jax.ShapeDtypeStruct((2, N_PAD), jnp.float32),
            jax.ShapeDtypeStruct((2, N_PAD), jnp.float32),
        ],
        mesh=_mesh(),
        scratch_types=[
            pltpu.VMEM((CPT, CHUNK), jnp.int32),
            pltpu.VMEM((CPT, CHUNK), jnp.int32),
            pltpu.VMEM((CHUNK,), jnp.float32),
            pltpu.VMEM((N_PER_TILE,), jnp.float32),
            pltpu.VMEM_SHARED((N_PAD,), jnp.float32),
            pltpu.VMEM_SHARED((N_PAD,), jnp.float32),
            pltpu.SemaphoreType.DMA,
            pltpu.SemaphoreType.DMA,
        ],
        compiler_params=pltpu.CompilerParams(use_tc_tiling_on_sc=False),
    )(src2d, dst2d)


def _sc_agg(y, src2d, dst2d):
    def body(y_h, src_h, dst_h, out_h, srcv, dstv, rows, zbuf, acc, gsem, ssem):
        c = lax.axis_index("c")
        s = lax.axis_index("s")
        tid = s * 2 + c

        def zb(i, _):
            zbuf[i // 4, pl.ds((i % 4) * 16, 16)] = jnp.zeros((16,), jnp.float32)
            return 0

        lax.fori_loop(0, 16 * 4, zb, 0)

        def zc(i, _):
            pltpu.async_copy(
                zbuf, acc.at[pl.ds(s * N_PER_TILE + i * 16, 16)], gsem.at[0])
            return 0

        lax.fori_loop(0, N_PER_TILE // 16, zc, 0)

        def zw(i, _):
            pltpu.make_async_copy(
                zbuf, acc.at[pl.ds(s * N_PER_TILE + i * 16, 16)], gsem.at[0]).wait()
            return 0

        lax.fori_loop(0, N_PER_TILE // 16, zw, 0)
        plsc.subcore_barrier()

        _stage_indices(src_h, dst_h, srcv, dstv, tid)
        nring = jnp.where(tid < 31, CPT, LAST_RING)

        for b in range(PF):
            pltpu.async_copy(y_h.at[srcv.at[b]], rows.at[b], gsem.at[b])

        def step(g, _):
            for b in range(NBUF):
                j = g * NBUF + b
                bn = (b + PF) % NBUF

                @pl.when(jnp.logical_and(j >= PF, j < nring - PF))
                def _wait_free():
                    pltpu.make_async_copy(
                        rows.at[bn], acc.at[dstv.at[j - PF]], ssem.at[bn]).wait()

                @pl.when(j < nring - PF)
                def _prefetch():
                    pltpu.async_copy(
                        y_h.at[srcv.at[j + PF]], rows.at[bn], gsem.at[bn])

                pltpu.make_async_copy(
                    y_h.at[srcv.at[j]], rows.at[b], gsem.at[b]).wait()
                pltpu.async_copy(
                    rows.at[b], acc.at[dstv.at[j]], ssem.at[b], add=True)
            return 0

        lax.fori_loop(0, nring // NBUF, step, 0)

        for i in range(NBUF):
            k = nring - NBUF + i
            pltpu.make_async_copy(
                rows.at[i], acc.at[dstv.at[k]], ssem.at[i]).wait()

        @pl.when(tid == 31)
        def _tail():
            for t in range(LAST_TAIL):
                pltpu.sync_copy(y_h.at[srcv.at[LAST_RING + t]], rows.at[0])
                pltpu.sync_copy(rows.at[0], acc.at[dstv.at[LAST_RING + t]],
                                add=True)

        plsc.subcore_barrier()

        sl = pl.ds(s * N_PER_TILE, N_PER_TILE)
        pltpu.sync_copy(acc.at[sl], out_h.at[c, sl])

    return pl.kernel(
        body,
        out_type=jax.ShapeDtypeStruct((2, N_PAD, F_H), jnp.float32),
        mesh=_mesh(),
        scratch_types=[
            pltpu.VMEM((CPT, CHUNK), jnp.int32),
            pltpu.VMEM((CPT, CHUNK), jnp.int32),
            pltpu.VMEM((NBUF, CHUNK, F_H), jnp.float32),
            pltpu.VMEM((16, F_H), jnp.float32),
            pltpu.VMEM_SHARED((N_PAD, F_H), jnp.float32),
            pltpu.SemaphoreType.DMA((NBUF,)),
            pltpu.SemaphoreType.DMA((NBUF,)),
        ],
        compiler_params=pltpu.CompilerParams(use_tc_tiling_on_sc=False),
    )(y, src2d, dst2d)


def _tc_matmul(x, W1):
    def body(x_ref, w_ref, o_ref):
        o_ref[...] = jnp.dot(x_ref[...], w_ref[...],
                             preferred_element_type=jnp.float32)

    return pl.pallas_call(
        body,
        grid=(N // ROW_BLK,),
        in_specs=[
            pl.BlockSpec((ROW_BLK, F_IN), lambda i: (i, 0)),
            pl.BlockSpec((F_IN, F_H), lambda i: (0, 0)),
        ],
        out_specs=pl.BlockSpec((ROW_BLK, F_H), lambda i: (i, 0)),
        out_shape=jax.ShapeDtypeStruct((N, F_H), jnp.float32),
    )(x, W1)


def _tc_scale(z, dout):
    def body(z_ref, d_ref, o_ref):
        ns = lax.rsqrt(jnp.maximum(d_ref[0] + d_ref[1], 1.0))
        o_ref[...] = z_ref[...] * ns

    return pl.pallas_call(
        body,
        grid=(N // ROW_BLK,),
        in_specs=[
            pl.BlockSpec((ROW_BLK, F_H), lambda i: (i, 0)),
            pl.BlockSpec((2, ROW_BLK, 1), lambda i: (0, i, 0)),
        ],
        out_specs=pl.BlockSpec((ROW_BLK, F_H), lambda i: (i, 0)),
        out_shape=jax.ShapeDtypeStruct((N, F_H), jnp.float32),
    )(z, dout)


def _tc_mid(agg, din, dout, b1, W2):
    def body(a_ref, i_ref, o_ref2, br, w_ref, o_ref):
        nd = lax.rsqrt(jnp.maximum(i_ref[0] + i_ref[1], 1.0))
        h = jnp.maximum((a_ref[0] + a_ref[1]) * nd + br[...], 0.0)
        ns = lax.rsqrt(jnp.maximum(o_ref2[0] + o_ref2[1], 1.0))
        o_ref[...] = jnp.dot(h * ns, w_ref[...],
                             preferred_element_type=jnp.float32)

    return pl.pallas_call(
        body,
        grid=(N // ROW_BLK,),
        in_specs=[
            pl.BlockSpec((2, ROW_BLK, F_H), lambda i: (0, i, 0)),
            pl.BlockSpec((2, ROW_BLK, 1), lambda i: (0, i, 0)),
            pl.BlockSpec((2, ROW_BLK, 1), lambda i: (0, i, 0)),
            pl.BlockSpec((1, F_H), lambda i: (0, 0)),
            pl.BlockSpec((F_H, F_H), lambda i: (0, 0)),
        ],
        out_specs=pl.BlockSpec((ROW_BLK, F_H), lambda i: (i, 0)),
        out_shape=jax.ShapeDtypeStruct((N, F_H), jnp.float32),
    )(agg, din, dout, b1, W2)


def _tc_final(agg, din, b2):
    def body(a_ref, i_ref, br, o_ref):
        nd = lax.rsqrt(jnp.maximum(i_ref[0] + i_ref[1], 1.0))
        o_ref[...] = jnp.maximum((a_ref[0] + a_ref[1]) * nd + br[...], 0.0)

    return pl.pallas_call(
        body,
        grid=(N // ROW_BLK,),
        in_specs=[
            pl.BlockSpec((2, ROW_BLK, F_H), lambda i: (0, i, 0)),
            pl.BlockSpec((2, ROW_BLK, 1), lambda i: (0, i, 0)),
            pl.BlockSpec((1, F_H), lambda i: (0, 0)),
        ],
        out_specs=pl.BlockSpec((ROW_BLK, F_H), lambda i: (i, 0)),
        out_shape=jax.ShapeDtypeStruct((N, F_H), jnp.float32),
    )(agg, din, b2)


def kernel(inputs, edge_index, W1, b1, W2, b2):
    src2d = edge_index[0].reshape(NCHUNKS, CHUNK)
    dst2d = edge_index[1].reshape(NCHUNKS, CHUNK)

    degp_out, degp_in = _sc_degrees(src2d, dst2d)
    dout = degp_out.reshape(2, N_PAD, 1)
    din = degp_in.reshape(2, N_PAD, 1)

    z1 = _tc_matmul(inputs, W1)
    y1 = _tc_scale(z1, dout)
    agg1 = _sc_agg(y1, src2d, dst2d)
    y2 = _tc_mid(agg1, din, dout, b1.reshape(1, F_H), W2)
    agg2 = _sc_agg(y2, src2d, dst2d)
    return _tc_final(agg2, din, b2.reshape(1, F_H))


# speedup vs baseline: 1.2440x; 1.2440x over previous
import jax
import jax.numpy as jnp
from jax import lax
from jax.experimental import pallas as pl
from jax.experimental.pallas import tpu as pltpu
from jax.experimental.pallas import tpu_sc as plsc

N = 10000
EDGES = 320000
F_IN = 128
F_H = 64

N_PAD = 10240
N_PER_TILE = N_PAD // 16
CHUNK = 128
N_TILES = 32
NCHUNKS = EDGES // CHUNK
CPT = 80
LAST_RING = 16
LAST_TAIL = NCHUNKS - 31 * CPT - LAST_RING
ROW_BLK = 1000

NBUF = 8
PF = 4


def _mesh():
    return plsc.VectorSubcoreMesh(core_axis_name="c", subcore_axis_name="s")


def _stage_indices(src_h, dst_h, srcv, dstv, tid):
    @pl.when(tid < 31)
    def _full():
        pltpu.sync_copy(src_h.at[pl.ds(tid * CPT, CPT)], srcv.at[pl.ds(0, CPT)])
        pltpu.sync_copy(dst_h.at[pl.ds(tid * CPT, CPT)], dstv.at[pl.ds(0, CPT)])

    @pl.when(tid == 31)
    def _last():
        nlast = LAST_RING + LAST_TAIL
        pltpu.sync_copy(src_h.at[pl.ds(31 * CPT, nlast)], srcv.at[pl.ds(0, nlast)])
        pltpu.sync_copy(dst_h.at[pl.ds(31 * CPT, nlast)], dstv.at[pl.ds(0, nlast)])


def _sc_degrees(src2d, dst2d):
    def body(src_h, dst_h, dout_h, din_h, srcv, dstv, ones_v, zv, acc_o, acc_i,
             sem_a, sem_b):
        c = lax.axis_index("c")
        s = lax.axis_index("s")
        tid = s * 2 + c
        nt = jnp.where(tid < 31, CPT, LAST_RING + LAST_TAIL)

        def set_ones(i, _):
            ones_v[pl.ds(i * 16, 16)] = jnp.ones((16,), jnp.float32)
            return 0

        lax.fori_loop(0, CHUNK // 16, set_ones, 0)

        def set_zero(i, _):
            zv[pl.ds(i * 16, 16)] = jnp.zeros((16,), jnp.float32)
            return 0

        lax.fori_loop(0, N_PER_TILE // 16, set_zero, 0)

        sl = pl.ds(s * N_PER_TILE, N_PER_TILE)
        pltpu.sync_copy(zv, acc_o.at[sl])
        pltpu.sync_copy(zv, acc_i.at[sl])
        plsc.subcore_barrier()

        _stage_indices(src_h, dst_h, srcv, dstv, tid)

        def step(j, _):
            pltpu.async_copy(ones_v, acc_o.at[srcv.at[j]], sem_a, add=True)
            pltpu.async_copy(ones_v, acc_i.at[dstv.at[j]], sem_b, add=True)

            @pl.when(j >= 16)
            def _drain_old():
                pltpu.make_async_copy(ones_v, acc_o.at[srcv.at[j - 16]], sem_a).wait()
                pltpu.make_async_copy(ones_v, acc_i.at[dstv.at[j - 16]], sem_b).wait()

            return 0

        lax.fori_loop(0, nt, step, 0)

        def drain(i, _):
            pltpu.make_async_copy(ones_v, acc_o.at[srcv.at[nt - 16 + i]], sem_a).wait()
            pltpu.make_async_copy(ones_v, acc_i.at[dstv.at[nt - 16 + i]], sem_b).wait()
            return 0

        lax.fori_loop(0, 16, drain, 0)
        plsc.subcore_barrier()

        pltpu.sync_copy(acc_o.at[sl], dout_h.at[c, sl])
        pltpu.sync_copy(acc_i.at[sl], din_h.at[c, sl])

    return pl.kernel(
        body,
        out_type=[
            jax.ShapeDtypeStruct((2, N_PAD), jnp.float32),
            jax.ShapeDtypeStruct((2, N_PAD), jnp.float32),
        ],
        mesh=_mesh(),
        scratch_types=[
            pltpu.VMEM((CPT, CHUNK), jnp.int32),
            pltpu.VMEM((CPT, CHUNK), jnp.int32),
            pltpu.VMEM((CHUNK,), jnp.float32),
            pltpu.VMEM((N_PER_TILE,), jnp.float32),
            pltpu.VMEM_SHARED((N_PAD,), jnp.float32),
            pltpu.VMEM_SHARED((N_PAD,), jnp.float32),
            pltpu.SemaphoreType.DMA,
            pltpu.SemaphoreType.DMA,
        ],
    )(src2d, dst2d)


def _sc_agg(y, src2d, dst2d):
    def body(y_h, src_h, dst_h, out_h, srcv, dstv, rows, zbuf, acc, gsem, ssem):
        c = lax.axis_index("c")
        s = lax.axis_index("s")
        tid = s * 2 + c

        def zb(i, _):
            zbuf[i // 4, pl.ds((i % 4) * 16, 16)] = jnp.zeros((16,), jnp.float32)
            return 0

        lax.fori_loop(0, 16 * 4, zb, 0)

        def zc(i, _):
            pltpu.async_copy(
                zbuf, acc.at[pl.ds(s * N_PER_TILE + i * 16, 16)], gsem.at[0])
            return 0

        lax.fori_loop(0, N_PER_TILE // 16, zc, 0)

        def zw(i, _):
            pltpu.make_async_copy(
                zbuf, acc.at[pl.ds(s * N_PER_TILE + i * 16, 16)], gsem.at[0]).wait()
            return 0

        lax.fori_loop(0, N_PER_TILE // 16, zw, 0)
        plsc.subcore_barrier()

        _stage_indices(src_h, dst_h, srcv, dstv, tid)
        nring = jnp.where(tid < 31, CPT, LAST_RING)

        for b in range(PF):
            pltpu.async_copy(y_h.at[srcv.at[b]], rows.at[b], gsem.at[b])

        def step(g, _):
            for b in range(NBUF):
                j = g * NBUF + b
                bn = (b + PF) % NBUF

                @pl.when(jnp.logical_and(j >= PF, j < nring - PF))
                def _wait_free():
                    pltpu.make_async_copy(
                        rows.at[bn], acc.at[dstv.at[j - PF]], ssem.at[bn]).wait()

                @pl.when(j < nring - PF)
                def _prefetch():
                    pltpu.async_copy(
                        y_h.at[srcv.at[j + PF]], rows.at[bn], gsem.at[bn])

                pltpu.make_async_copy(
                    y_h.at[srcv.at[j]], rows.at[b], gsem.at[b]).wait()
                pltpu.async_copy(
                    rows.at[b], acc.at[dstv.at[j]], ssem.at[b], add=True)
            return 0

        lax.fori_loop(0, nring // NBUF, step, 0)

        for i in range(NBUF):
            k = nring - NBUF + i
            pltpu.make_async_copy(
                rows.at[i], acc.at[dstv.at[k]], ssem.at[i]).wait()

        @pl.when(tid == 31)
        def _tail():
            for t in range(LAST_TAIL):
                pltpu.sync_copy(y_h.at[srcv.at[LAST_RING + t]], rows.at[0])
                pltpu.sync_copy(rows.at[0], acc.at[dstv.at[LAST_RING + t]],
                                add=True)

        plsc.subcore_barrier()

        sl = pl.ds(s * N_PER_TILE, N_PER_TILE)
        pltpu.sync_copy(acc.at[sl], out_h.at[c, sl])

    return pl.kernel(
        body,
        out_type=jax.ShapeDtypeStruct((2, N_PAD, F_H), jnp.float32),
        mesh=_mesh(),
        scratch_types=[
            pltpu.VMEM((CPT, CHUNK), jnp.int32),
            pltpu.VMEM((CPT, CHUNK), jnp.int32),
            pltpu.VMEM((NBUF, CHUNK, F_H), jnp.float32),
            pltpu.VMEM((16, F_H), jnp.float32),
            pltpu.VMEM_SHARED((N_PAD, F_H), jnp.float32),
            pltpu.SemaphoreType.DMA((NBUF,)),
            pltpu.SemaphoreType.DMA((NBUF,)),
        ],
        compiler_params=pltpu.CompilerParams(use_tc_tiling_on_sc=False),
    )(y, src2d, dst2d)


NP2 = N // 2
PB = 1000


def _paired_norm(d0, d1):
    nsv = lax.rsqrt(jnp.maximum(d0 + d1, 1.0))
    lane = lax.broadcasted_iota(jnp.int32, (PB, 2 * F_H), 1)
    return jnp.where(lane < F_H, nsv[:, 0:1], nsv[:, 1:2])


def _tc_matmul(xp, W1bd):
    def body(x_ref, w_ref, o_ref):
        o_ref[...] = jnp.dot(x_ref[...], w_ref[...],
                             preferred_element_type=jnp.float32)

    return pl.pallas_call(
        body,
        grid=(NP2 // PB,),
        in_specs=[
            pl.BlockSpec((PB, 2 * F_IN), lambda i: (i, 0)),
            pl.BlockSpec((2 * F_IN, 2 * F_H), lambda i: (0, 0)),
        ],
        out_specs=pl.BlockSpec((PB, 2 * F_H), lambda i: (i, 0)),
        out_shape=jax.ShapeDtypeStruct((NP2, 2 * F_H), jnp.float32),
    )(xp, W1bd)


def _tc_scale(zp, doutp):
    def body(z_ref, d_ref, o_ref):
        o_ref[...] = z_ref[...] * _paired_norm(d_ref[0], d_ref[1])

    return pl.pallas_call(
        body,
        grid=(NP2 // PB,),
        in_specs=[
            pl.BlockSpec((PB, 2 * F_H), lambda i: (i, 0)),
            pl.BlockSpec((2, PB, 2), lambda i: (0, i, 0)),
        ],
        out_specs=pl.BlockSpec((PB, 2 * F_H), lambda i: (i, 0)),
        out_shape=jax.ShapeDtypeStruct((NP2, 2 * F_H), jnp.float32),
    )(zp, doutp)


def _tc_mid(aggp, dinp, doutp, b1p, W2bd):
    def body(a_ref, i_ref, o2_ref, br, w_ref, o_ref):
        nd = _paired_norm(i_ref[0], i_ref[1])
        h = jnp.maximum((a_ref[0] + a_ref[1]) * nd + br[...], 0.0)
        ns = _paired_norm(o2_ref[0], o2_ref[1])
        o_ref[...] = jnp.dot(h * ns, w_ref[...],
                             preferred_element_type=jnp.float32)

    return pl.pallas_call(
        body,
        grid=(NP2 // PB,),
        in_specs=[
            pl.BlockSpec((2, PB, 2 * F_H), lambda i: (0, i, 0)),
            pl.BlockSpec((2, PB, 2), lambda i: (0, i, 0)),
            pl.BlockSpec((2, PB, 2), lambda i: (0, i, 0)),
            pl.BlockSpec((1, 2 * F_H), lambda i: (0, 0)),
            pl.BlockSpec((2 * F_H, 2 * F_H), lambda i: (0, 0)),
        ],
        out_specs=pl.BlockSpec((PB, 2 * F_H), lambda i: (i, 0)),
        out_shape=jax.ShapeDtypeStruct((NP2, 2 * F_H), jnp.float32),
    )(aggp, dinp, doutp, b1p, W2bd)


def _tc_final(aggp, dinp, b2p):
    def body(a_ref, i_ref, br, o_ref):
        nd = _paired_norm(i_ref[0], i_ref[1])
        o_ref[...] = jnp.maximum((a_ref[0] + a_ref[1]) * nd + br[...], 0.0)

    return pl.pallas_call(
        body,
        grid=(NP2 // PB,),
        in_specs=[
            pl.BlockSpec((2, PB, 2 * F_H), lambda i: (0, i, 0)),
            pl.BlockSpec((2, PB, 2), lambda i: (0, i, 0)),
            pl.BlockSpec((1, 2 * F_H), lambda i: (0, 0)),
        ],
        out_specs=pl.BlockSpec((PB, 2 * F_H), lambda i: (i, 0)),
        out_shape=jax.ShapeDtypeStruct((NP2, 2 * F_H), jnp.float32),
    )(aggp, dinp, b2p)


def _blockdiag(W):
    k, m = W.shape
    top = jnp.concatenate([W, jnp.zeros((k, m), jnp.float32)], axis=1)
    bot = jnp.concatenate([jnp.zeros((k, m), jnp.float32), W], axis=1)
    return jnp.concatenate([top, bot], axis=0)


def kernel(inputs, edge_index, W1, b1, W2, b2):
    src2d = edge_index[0].reshape(NCHUNKS, CHUNK)
    dst2d = edge_index[1].reshape(NCHUNKS, CHUNK)

    degp_out, degp_in = _sc_degrees(src2d, dst2d)
    doutp = degp_out.reshape(2, N_PAD // 2, 2)
    dinp = degp_in.reshape(2, N_PAD // 2, 2)

    xp = inputs.reshape(NP2, 2 * F_IN)
    z1p = _tc_matmul(xp, _blockdiag(W1))
    y1p = _tc_scale(z1p, doutp)
    agg1 = _sc_agg(y1p.reshape(N, F_H), src2d, dst2d)
    y2p = _tc_mid(agg1.reshape(2, N_PAD // 2, 2 * F_H), dinp, doutp,
                  jnp.concatenate([b1, b1]).reshape(1, 2 * F_H),
                  _blockdiag(W2))
    agg2 = _sc_agg(y2p.reshape(N, F_H), src2d, dst2d)
    outp = _tc_final(agg2.reshape(2, N_PAD // 2, 2 * F_H), dinp,
                     jnp.concatenate([b2, b2]).reshape(1, 2 * F_H))
    return outp.reshape(N, F_H)


# speedup vs baseline: 1.3239x; 1.0642x over previous
import jax
import jax.numpy as jnp
from jax import lax
from jax.experimental import pallas as pl
from jax.experimental.pallas import tpu as pltpu
from jax.experimental.pallas import tpu_sc as plsc

N = 10000
EDGES = 320000
F_IN = 128
F_H = 64

N_PAD = 10240
N_PER_TILE = N_PAD // 16
CHUNK = 128
N_TILES = 32
NCHUNKS = EDGES // CHUNK
CPT = 80
LAST_RING = 16
LAST_TAIL = NCHUNKS - 31 * CPT - LAST_RING
ROW_BLK = 1000

NBUF = 8
PF = 4


def _mesh():
    return plsc.VectorSubcoreMesh(core_axis_name="c", subcore_axis_name="s")


def _stage_indices(e_h, srcv, dstv, tid):
    @pl.when(tid < 31)
    def _full():
        pltpu.sync_copy(e_h.at[0, pl.ds(tid * CPT, CPT)], srcv.at[pl.ds(0, CPT)])
        pltpu.sync_copy(e_h.at[1, pl.ds(tid * CPT, CPT)], dstv.at[pl.ds(0, CPT)])

    @pl.when(tid == 31)
    def _last():
        nlast = LAST_RING + LAST_TAIL
        pltpu.sync_copy(e_h.at[0, pl.ds(31 * CPT, nlast)], srcv.at[pl.ds(0, nlast)])
        pltpu.sync_copy(e_h.at[1, pl.ds(31 * CPT, nlast)], dstv.at[pl.ds(0, nlast)])


def _sc_degrees(e3):
    def body(e_h, dout_h, din_h, srcv, dstv, ones_v, zv, acc_o, acc_i,
             sem_a, sem_b):
        c = lax.axis_index("c")
        s = lax.axis_index("s")
        tid = s * 2 + c
        nt = jnp.where(tid < 31, CPT, LAST_RING + LAST_TAIL)

        def set_ones(i, _):
            ones_v[pl.ds(i * 16, 16)] = jnp.ones((16,), jnp.float32)
            return 0

        lax.fori_loop(0, CHUNK // 16, set_ones, 0)

        def set_zero(i, _):
            zv[pl.ds(i * 16, 16)] = jnp.zeros((16,), jnp.float32)
            return 0

        lax.fori_loop(0, N_PER_TILE // 16, set_zero, 0)

        sl = pl.ds(s * N_PER_TILE, N_PER_TILE)
        pltpu.sync_copy(zv, acc_o.at[sl])
        pltpu.sync_copy(zv, acc_i.at[sl])
        plsc.subcore_barrier()

        _stage_indices(e_h, srcv, dstv, tid)

        def step(j, _):
            pltpu.async_copy(ones_v, acc_o.at[srcv.at[j]], sem_a, add=True)
            pltpu.async_copy(ones_v, acc_i.at[dstv.at[j]], sem_b, add=True)

            @pl.when(j >= 16)
            def _drain_old():
                pltpu.make_async_copy(ones_v, acc_o.at[srcv.at[j - 16]], sem_a).wait()
                pltpu.make_async_copy(ones_v, acc_i.at[dstv.at[j - 16]], sem_b).wait()

            return 0

        lax.fori_loop(0, nt, step, 0)

        def drain(i, _):
            pltpu.make_async_copy(ones_v, acc_o.at[srcv.at[nt - 16 + i]], sem_a).wait()
            pltpu.make_async_copy(ones_v, acc_i.at[dstv.at[nt - 16 + i]], sem_b).wait()
            return 0

        lax.fori_loop(0, 16, drain, 0)
        plsc.subcore_barrier()

        pltpu.sync_copy(acc_o.at[sl], dout_h.at[c, sl])
        pltpu.sync_copy(acc_i.at[sl], din_h.at[c, sl])

    return pl.kernel(
        body,
        out_type=[
            jax.ShapeDtypeStruct((2, N_PAD), jnp.float32),
            jax.ShapeDtypeStruct((2, N_PAD), jnp.float32),
        ],
        mesh=_mesh(),
        scratch_types=[
            pltpu.VMEM((CPT, CHUNK), jnp.int32),
            pltpu.VMEM((CPT, CHUNK), jnp.int32),
            pltpu.VMEM((CHUNK,), jnp.float32),
            pltpu.VMEM((N_PER_TILE,), jnp.float32),
            pltpu.VMEM_SHARED((N_PAD,), jnp.float32),
            pltpu.VMEM_SHARED((N_PAD,), jnp.float32),
            pltpu.SemaphoreType.DMA,
            pltpu.SemaphoreType.DMA,
        ],
        compiler_params=pltpu.CompilerParams(use_tc_tiling_on_sc=False),
    )(e3)


def _sc_agg(y, e3):
    def body(y_h, e_h, out_h, srcv, dstv, rows, zbuf, acc, gsem, ssem):
        c = lax.axis_index("c")
        s = lax.axis_index("s")
        tid = s * 2 + c

        def zb(i, _):
            zbuf[i // 4, pl.ds((i % 4) * 16, 16)] = jnp.zeros((16,), jnp.float32)
            return 0

        lax.fori_loop(0, 16 * 4, zb, 0)

        def zc(i, _):
            pltpu.async_copy(
                zbuf, acc.at[pl.ds(s * N_PER_TILE + i * 16, 16)], gsem.at[0])
            return 0

        lax.fori_loop(0, N_PER_TILE // 16, zc, 0)

        def zw(i, _):
            pltpu.make_async_copy(
                zbuf, acc.at[pl.ds(s * N_PER_TILE + i * 16, 16)], gsem.at[0]).wait()
            return 0

        lax.fori_loop(0, N_PER_TILE // 16, zw, 0)
        plsc.subcore_barrier()

        _stage_indices(e_h, srcv, dstv, tid)
        nring = jnp.where(tid < 31, CPT, LAST_RING)

        for b in range(PF):
            pltpu.async_copy(y_h.at[srcv.at[b]], rows.at[b], gsem.at[b])

        def step(g, _):
            for b in range(NBUF):
                j = g * NBUF + b
                bn = (b + PF) % NBUF

                @pl.when(jnp.logical_and(j >= PF, j < nring - PF))
                def _wait_free():
                    pltpu.make_async_copy(
                        rows.at[bn], acc.at[dstv.at[j - PF]], ssem.at[bn]).wait()

                @pl.when(j < nring - PF)
                def _prefetch():
                    pltpu.async_copy(
                        y_h.at[srcv.at[j + PF]], rows.at[bn], gsem.at[bn])

                pltpu.make_async_copy(
                    y_h.at[srcv.at[j]], rows.at[b], gsem.at[b]).wait()
                pltpu.async_copy(
                    rows.at[b], acc.at[dstv.at[j]], ssem.at[b], add=True)
            return 0

        lax.fori_loop(0, nring // NBUF, step, 0)

        for i in range(NBUF):
            k = nring - NBUF + i
            pltpu.make_async_copy(
                rows.at[i], acc.at[dstv.at[k]], ssem.at[i]).wait()

        @pl.when(tid == 31)
        def _tail():
            for t in range(LAST_TAIL):
                pltpu.sync_copy(y_h.at[srcv.at[LAST_RING + t]], rows.at[0])
                pltpu.sync_copy(rows.at[0], acc.at[dstv.at[LAST_RING + t]],
                                add=True)

        plsc.subcore_barrier()

        sl = pl.ds(s * N_PER_TILE, N_PER_TILE)
        pltpu.sync_copy(acc.at[sl], out_h.at[c, sl])

    return pl.kernel(
        body,
        out_type=jax.ShapeDtypeStruct((2, N_PAD, F_H), jnp.float32),
        mesh=_mesh(),
        scratch_types=[
            pltpu.VMEM((CPT, CHUNK), jnp.int32),
            pltpu.VMEM((CPT, CHUNK), jnp.int32),
            pltpu.VMEM((NBUF, CHUNK, F_H), jnp.float32),
            pltpu.VMEM((16, F_H), jnp.float32),
            pltpu.VMEM_SHARED((N_PAD, F_H), jnp.float32),
            pltpu.SemaphoreType.DMA((NBUF,)),
            pltpu.SemaphoreType.DMA((NBUF,)),
        ],
        compiler_params=pltpu.CompilerParams(use_tc_tiling_on_sc=False),
    )(y, e3)


NP2 = N // 2
PB = 1000


def _paired_norm(d0, d1):
    nsv = lax.rsqrt(jnp.maximum(d0 + d1, 1.0))
    lane = lax.broadcasted_iota(jnp.int32, (PB, 2 * F_H), 1)
    return jnp.where(lane < F_H, nsv[:, 0:1], nsv[:, 1:2])


def _tc_matmul(xp, W1bd):
    def body(x_ref, w_ref, o_ref):
        o_ref[...] = jnp.dot(x_ref[...], w_ref[...],
                             preferred_element_type=jnp.float32)

    return pl.pallas_call(
        body,
        grid=(NP2 // PB,),
        in_specs=[
            pl.BlockSpec((PB, 2 * F_IN), lambda i: (i, 0)),
            pl.BlockSpec((2 * F_IN, 2 * F_H), lambda i: (0, 0)),
        ],
        out_specs=pl.BlockSpec((PB, 2 * F_H), lambda i: (i, 0)),
        out_shape=jax.ShapeDtypeStruct((NP2, 2 * F_H), jnp.float32),
    )(xp, W1bd)


def _tc_scale(zp, doutp):
    def body(z_ref, d_ref, o_ref):
        o_ref[...] = z_ref[...] * _paired_norm(d_ref[0], d_ref[1])

    return pl.pallas_call(
        body,
        grid=(NP2 // PB,),
        in_specs=[
            pl.BlockSpec((PB, 2 * F_H), lambda i: (i, 0)),
            pl.BlockSpec((2, PB, 2), lambda i: (0, i, 0)),
        ],
        out_specs=pl.BlockSpec((PB, 2 * F_H), lambda i: (i, 0)),
        out_shape=jax.ShapeDtypeStruct((NP2, 2 * F_H), jnp.float32),
    )(zp, doutp)


def _tc_mid(aggp, dinp, doutp, b1p, W2bd):
    def body(a_ref, i_ref, o2_ref, br, w_ref, o_ref):
        nd = _paired_norm(i_ref[0], i_ref[1])
        h = jnp.maximum((a_ref[0] + a_ref[1]) * nd + br[...], 0.0)
        ns = _paired_norm(o2_ref[0], o2_ref[1])
        o_ref[...] = jnp.dot(h * ns, w_ref[...],
                             preferred_element_type=jnp.float32)

    return pl.pallas_call(
        body,
        grid=(NP2 // PB,),
        in_specs=[
            pl.BlockSpec((2, PB, 2 * F_H), lambda i: (0, i, 0)),
            pl.BlockSpec((2, PB, 2), lambda i: (0, i, 0)),
            pl.BlockSpec((2, PB, 2), lambda i: (0, i, 0)),
            pl.BlockSpec((1, 2 * F_H), lambda i: (0, 0)),
            pl.BlockSpec((2 * F_H, 2 * F_H), lambda i: (0, 0)),
        ],
        out_specs=pl.BlockSpec((PB, 2 * F_H), lambda i: (i, 0)),
        out_shape=jax.ShapeDtypeStruct((NP2, 2 * F_H), jnp.float32),
    )(aggp, dinp, doutp, b1p, W2bd)


def _tc_final(aggp, dinp, b2p):
    def body(a_ref, i_ref, br, o_ref):
        nd = _paired_norm(i_ref[0], i_ref[1])
        o_ref[...] = jnp.maximum((a_ref[0] + a_ref[1]) * nd + br[...], 0.0)

    return pl.pallas_call(
        body,
        grid=(NP2 // PB,),
        in_specs=[
            pl.BlockSpec((2, PB, 2 * F_H), lambda i: (0, i, 0)),
            pl.BlockSpec((2, PB, 2), lambda i: (0, i, 0)),
            pl.BlockSpec((1, 2 * F_H), lambda i: (0, 0)),
        ],
        out_specs=pl.BlockSpec((PB, 2 * F_H), lambda i: (i, 0)),
        out_shape=jax.ShapeDtypeStruct((NP2, 2 * F_H), jnp.float32),
    )(aggp, dinp, b2p)


def _blockdiag(W):
    k, m = W.shape
    top = jnp.concatenate([W, jnp.zeros((k, m), jnp.float32)], axis=1)
    bot = jnp.concatenate([jnp.zeros((k, m), jnp.float32), W], axis=1)
    return jnp.concatenate([top, bot], axis=0)


def kernel(inputs, edge_index, W1, b1, W2, b2):
    e3 = edge_index.reshape(2, NCHUNKS, CHUNK)

    degp_out, degp_in = _sc_degrees(e3)
    doutp = degp_out.reshape(2, N_PAD // 2, 2)
    dinp = degp_in.reshape(2, N_PAD // 2, 2)

    xp = inputs.reshape(NP2, 2 * F_IN)
    z1p = _tc_matmul(xp, _blockdiag(W1))
    y1p = _tc_scale(z1p, doutp)
    agg1 = _sc_agg(y1p.reshape(N, F_H), e3)
    y2p = _tc_mid(agg1.reshape(2, N_PAD // 2, 2 * F_H), dinp, doutp,
                  jnp.concatenate([b1, b1]).reshape(1, 2 * F_H),
                  _blockdiag(W2))
    agg2 = _sc_agg(y2p.reshape(N, F_H), e3)
    outp = _tc_final(agg2.reshape(2, N_PAD // 2, 2 * F_H), dinp,
                     jnp.concatenate([b2, b2]).reshape(1, 2 * F_H))
    return outp.reshape(N, F_H)
